# baseline (speedup 1.0000x reference)
import functools

import jax
import jax.numpy as jnp
from jax import lax
from jax.experimental import pallas as pl
from jax.experimental.pallas import tpu as pltpu
from jax.experimental.pallas import tpu_sc as plsc

B = 1024
S = 26
C = 1000
ROW = S * C
N = B * ROW
LANES = 16

_info = plsc.get_sparse_core_info()
NW = _info.num_cores * _info.num_subcores
ROWS_PER_W = B // NW
IDX_PER_W = ROWS_PER_W * 2 * LANES
KCHUNK = 128
NSTREAM = IDX_PER_W // KCHUNK

_mesh = plsc.VectorSubcoreMesh(core_axis_name="c", subcore_axis_name="s")


@functools.partial(
    pl.kernel,
    mesh=_mesh,
    compiler_params=pltpu.CompilerParams(needs_layout_passes=False),
    scratch_types=[
        pltpu.VMEM((ROWS_PER_W, S), jnp.int32),
        pltpu.VMEM((NSTREAM, KCHUNK), jnp.int32),
        pltpu.VMEM((KCHUNK,), jnp.float32),
        pltpu.SemaphoreType.DMA,
    ],
)
def _scatter_sc(batch_hbm, out_hbm, idx_v, pos_v, ones_v, sem):
    wid = lax.axis_index("s") * _info.num_cores + lax.axis_index("c")
    base = wid * ROWS_PER_W
    pltpu.sync_copy(batch_hbm.at[pl.ds(base, ROWS_PER_W)], idx_v)

    ones_f = jnp.ones((LANES,), jnp.float32)
    iota = lax.iota(jnp.int32, LANES)
    off_lo = iota * C
    off_hi = (iota + (S - LANES)) * C
    for u in range(KCHUNK // LANES):
        ones_v[pl.ds(u * LANES, LANES)] = ones_f

    for r in range(ROWS_PER_W):
        rowbase = (base + r) * ROW
        p_lo = idx_v[r, pl.ds(0, LANES)] + off_lo + rowbase
        p_hi = idx_v[r, pl.ds(S - LANES, LANES)] + off_hi + rowbase
        j, o = divmod(r * 2 * LANES, KCHUNK)
        pos_v[j, pl.ds(o, LANES)] = p_lo
        pos_v[j, pl.ds(o + LANES, LANES)] = p_hi

    copies = []
    for j in range(NSTREAM):
        copies.append(
            pltpu.async_copy(ones_v, out_hbm.at[pos_v.at[j]], sem)
        )
    for cp in copies:
        cp.wait()


def kernel(batch, lookup):
    del lookup
    batch = jnp.asarray(batch, jnp.int32)
    zero = (batch[0, 0] * 0).astype(jnp.float32)
    out_ref = jax.new_ref(jnp.zeros((N,), jnp.float32) + zero)
    _scatter_sc(batch, out_ref)
    return out_ref[...].reshape(B, ROW)


# speedup vs baseline: 2.4083x; 2.4083x over previous
import jax
import jax.numpy as jnp
from jax import lax
from jax.experimental import pallas as pl

B = 1024
S = 26
C = 1000
ROW = S * C
BR = 64


def _body(batch_ref, g_ref, m_ref, out_ref):
    idxf = batch_ref[...].astype(jnp.float32)
    t = jnp.dot(idxf, g_ref[...], preferred_element_type=jnp.float32)
    m = m_ref[...]
    out_ref[...] = jnp.where(t == m, 1.0, 0.0)


@jax.jit
def _onehot_tc(batch):
    cols = jnp.arange(ROW, dtype=jnp.int32)
    g = (cols[None, :] // C == jnp.arange(S, dtype=jnp.int32)[:, None])
    g = g.astype(jnp.float32)
    m = (cols % C).astype(jnp.float32)[None, :]
    return pl.pallas_call(
        _body,
        out_shape=jax.ShapeDtypeStruct((B, ROW), jnp.float32),
        grid=(B // BR,),
        in_specs=[
            pl.BlockSpec((BR, S), lambda i: (i, 0)),
            pl.BlockSpec((S, ROW), lambda i: (0, 0)),
            pl.BlockSpec((1, ROW), lambda i: (0, 0)),
        ],
        out_specs=pl.BlockSpec((BR, ROW), lambda i: (i, 0)),
    )(batch, g, m)


def kernel(batch, lookup):
    del lookup
    return _onehot_tc(jnp.asarray(batch, jnp.int32))
